# baseline (speedup 1.0000x reference)
import functools

import jax
import jax.numpy as jnp
from jax import lax
from jax.experimental import pallas as pl
from jax.experimental.pallas import tpu as pltpu
from jax.experimental.pallas import tpu_sc as plsc

N = 10000
E = 320000
T = 8
C = 128
NP = 10240
NC = 2
NS = 16
RPT = NP // NS
TPS = T // NC

NB = 16
NSRC = 32
EPW = E // NSRC
RCAP = 10272
SCH = 1024
K = 32
DL = 16

_f32 = jnp.float32
_i32 = jnp.int32

_sc_mesh = plsc.VectorSubcoreMesh(
    core_axis_name="c", subcore_axis_name="s", num_cores=NC, num_subcores=NS)


def _extract(v0, v1, j):
    return v0[j] if j < 16 else v1[j - 16]


def _dyn_lane(v0, v1, j):
    acc = v0[0]
    for k in range(1, 16):
        acc = jnp.where(j == k, v0[k], acc)
    for k in range(16):
        acc = jnp.where(j == k + 16, v1[k], acc)
    return acc


@functools.partial(
    pl.kernel,
    out_type=jax.ShapeDtypeStruct((NC * NP, DL), _f32),
    mesh=_sc_mesh,
    scratch_types=[
        pltpu.VMEM((NSRC,), _i32),
        pltpu.VMEM((SCH,), _i32),
        pltpu.VMEM((SCH,), _f32),
        pltpu.VMEM((RPT, DL), _f32),
    ],
)
def _deg_kernel(bcl, bw, counts2, out, cntv, scl, sw, dacc):
    cid = lax.axis_index("c")
    b = lax.axis_index("s")
    pltpu.sync_copy(counts2.at[b], cntv)
    cv0 = cntv[pl.ds(0, 16)]
    cv1 = cntv[pl.ds(16, 16)]

    zv = jnp.zeros((DL,), _f32)

    def zr(i, carry):
        dacc[i, :] = zv
        return carry

    lax.fori_loop(0, RPT, zr, 0)

    cvsel = jnp.where(cid == 0, cv0, cv1)

    def sloop(sl, carry):
        s = cid * (NSRC // NC) + sl
        cnt = _dyn_lane(cvsel, cvsel, sl)
        roff = (b * NSRC + s) * RCAP
        nstage = (cnt + (SCH - 1)) // SCH

        def stage(si, carry1):
            off = roff + si * SCH
            pltpu.sync_copy(bcl.at[pl.ds(off, SCH)], scl)
            pltpu.sync_copy(bw.at[pl.ds(off, SCH)], sw)
            rem = jnp.minimum(cnt - si * SCH, SCH)
            nin = (rem + (K - 1)) // K

            def chunk(ci, carry2):
                base = ci * K
                c0 = scl[pl.ds(base, 16)]
                c1 = scl[pl.ds(base + 16, 16)]
                w0 = sw[pl.ds(base, 16)]
                w1 = sw[pl.ds(base + 16, 16)]
                for j in range(K):
                    cl = _extract(c0, c1, j)
                    wj = _extract(w0, w1, j)
                    dacc[cl, :] = dacc[cl, :] + jnp.full((DL,), wj, _f32)
                return carry2

            lax.fori_loop(0, nin, chunk, 0)
            return carry1

        lax.fori_loop(0, nstage, stage, 0)
        return carry

    lax.fori_loop(0, NSRC // NC, sloop, 0)

    pltpu.sync_copy(dacc, out.at[pl.ds(cid * NP + b * RPT, RPT)])


@functools.partial(
    pl.kernel,
    out_type=jax.ShapeDtypeStruct((T * NP, C), _f32),
    mesh=_sc_mesh,
    scratch_types=[
        pltpu.VMEM((NSRC,), _i32),
        pltpu.VMEM((SCH,), _i32),
        pltpu.VMEM((SCH,), _i32),
        pltpu.VMEM((SCH,), _f32),
        pltpu.VMEM((K,), _i32),
        pltpu.VMEM((K,), _i32),
        pltpu.VMEM((K, C), _f32),
        pltpu.VMEM((K, C), _f32),
        pltpu.VMEM((RPT, C), _f32),
        pltpu.SemaphoreType.DMA,
        pltpu.SemaphoreType.DMA,
    ],
)
def _spmm_kernel(hpp, brow, bcl, bw, counts2, out,
                 cntv, srow, scl, sw, idxa, idxb, gbufa, gbufb, acc,
                 sema, semb):
    cid = lax.axis_index("c")
    b = lax.axis_index("s")
    pltpu.sync_copy(counts2.at[b], cntv)
    cv0 = cntv[pl.ds(0, 16)]
    cv1 = cntv[pl.ds(16, 16)]

    zv = jnp.zeros((16,), _f32)

    def tloop(tl, tcarry):
        t = cid * TPS + tl
        toff = t * NP

        def zr(i, carry):
            for v in range(C // 16):
                acc[i, pl.ds(v * 16, 16)] = zv
            return carry

        lax.fori_loop(0, RPT, zr, 0)

        def sloop(s, carry):
            cnt = _dyn_lane(cv0, cv1, s)
            roff = (b * NSRC + s) * RCAP
            nstage = (cnt + (SCH - 1)) // SCH

            def build(idx, ci):
                base = ci * K
                idx[pl.ds(0, 16)] = srow[pl.ds(base, 16)] + toff
                idx[pl.ds(16, 16)] = srow[pl.ds(base + 16, 16)] + toff

            def process(gbuf, ci):
                base = ci * K
                c0 = scl[pl.ds(base, 16)]
                c1 = scl[pl.ds(base + 16, 16)]
                w0 = sw[pl.ds(base, 16)]
                w1 = sw[pl.ds(base + 16, 16)]
                for j in range(K):
                    cl = _extract(c0, c1, j)
                    sv = jnp.full((16,), _extract(w0, w1, j), _f32)
                    for v in range(C // 16):
                        acc[cl, pl.ds(v * 16, 16)] = (
                            acc[cl, pl.ds(v * 16, 16)]
                            + gbuf[j, pl.ds(v * 16, 16)] * sv)

            def stage(si, carry1):
                off = roff + si * SCH
                pltpu.sync_copy(brow.at[pl.ds(off, SCH)], srow)
                pltpu.sync_copy(bcl.at[pl.ds(off, SCH)], scl)
                pltpu.sync_copy(bw.at[pl.ds(off, SCH)], sw)
                rem = jnp.minimum(cnt - si * SCH, SCH)
                nin = (rem + (K - 1)) // K

                @pl.when(nin > 0)
                def _prime():
                    build(idxa, 0)
                    pltpu.async_copy(hpp.at[idxa], gbufa, sema)

                def pair(pi, carry2):
                    ci0 = 2 * pi
                    ci1 = ci0 + 1

                    @pl.when(ci1 < nin)
                    def _startb():
                        build(idxb, ci1)
                        pltpu.async_copy(hpp.at[idxb], gbufb, semb)

                    pltpu.make_async_copy(hpp.at[idxa], gbufa, sema).wait()
                    process(gbufa, ci0)

                    @pl.when(ci0 + 2 < nin)
                    def _starta():
                        build(idxa, ci0 + 2)
                        pltpu.async_copy(hpp.at[idxa], gbufa, sema)

                    @pl.when(ci1 < nin)
                    def _drainb():
                        pltpu.make_async_copy(hpp.at[idxb], gbufb, semb).wait()
                        process(gbufb, ci1)

                    return carry2

                lax.fori_loop(0, (nin + 1) // 2, pair, 0)
                return carry1

            lax.fori_loop(0, nstage, stage, 0)
            return carry

        lax.fori_loop(0, NSRC, sloop, 0)
        pltpu.sync_copy(acc, out.at[pl.ds(toff + b * RPT, RPT)])
        return tcarry

    lax.fori_loop(0, TPS, tloop, 0)


BN_A = 1024


def _mm_body(x_ref, wc_ref, wg_ref, bc_ref, degp_ref, out_ref):
    xb = x_ref[...]
    wg_t = wg_ref[...].T
    m = [jnp.dot(wc_ref[k].T, wg_t, preferred_element_type=_f32)
         for k in range(3)]
    hb = jnp.dot(bc_ref[...], wg_t, preferred_element_type=_f32)
    pb = degp_ref[...]
    dis = lax.rsqrt(1.0 + pb[0, :, 0:1] + pb[1, :, 0:1])
    for t in range(T):
        acc = jnp.dot(xb[t], m[1], preferred_element_type=_f32) + hb
        if t > 0:
            acc = acc + jnp.dot(xb[t - 1], m[0], preferred_element_type=_f32)
        if t < T - 1:
            acc = acc + jnp.dot(xb[t + 1], m[2], preferred_element_type=_f32)
        out_ref[t] = acc * dis


_mm_call = pl.pallas_call(
    _mm_body,
    grid=(NP // BN_A,),
    in_specs=[
        pl.BlockSpec((T, BN_A, C), lambda i: (0, i, 0)),
        pl.BlockSpec((3, C, C), lambda i: (0, 0, 0)),
        pl.BlockSpec((C, C), lambda i: (0, 0)),
        pl.BlockSpec((1, C), lambda i: (0, 0)),
        pl.BlockSpec((2, BN_A, DL), lambda i: (0, i, 0)),
    ],
    out_specs=pl.BlockSpec((T, BN_A, C), lambda i: (0, i, 0)),
    out_shape=jax.ShapeDtypeStruct((T, NP, C), _f32),
)


def _bn_body(s_ref, h_ref, degp_ref, gamma_ref, beta_ref, out_ref):
    sb = s_ref[0]
    hb = h_ref[0]
    pb = degp_ref[...]
    dis = lax.rsqrt(1.0 + pb[0, :, 0:1] + pb[1, :, 0:1])
    o = dis * (sb + hb)
    mask = lax.broadcasted_iota(_i32, (NP, 1), 0) < N
    om = jnp.where(mask, o, 0.0)
    mu = jnp.sum(om, axis=0, keepdims=True) * (1.0 / N)
    d = jnp.where(mask, o - mu, 0.0)
    var = jnp.sum(d * d, axis=0, keepdims=True) * (1.0 / N)
    scale = gamma_ref[...] * lax.rsqrt(var + 1e-5)
    out_ref[0] = jnp.maximum((o - mu) * scale + beta_ref[...], 0.0)


_bn_call = pl.pallas_call(
    _bn_body,
    grid=(T,),
    in_specs=[
        pl.BlockSpec((1, NP, C), lambda t: (t, 0, 0)),
        pl.BlockSpec((1, NP, C), lambda t: (t, 0, 0)),
        pl.BlockSpec((2, NP, DL), lambda t: (0, 0, 0)),
        pl.BlockSpec((1, C), lambda t: (0, 0)),
        pl.BlockSpec((1, C), lambda t: (0, 0)),
    ],
    out_specs=pl.BlockSpec((1, NP, C), lambda t: (t, 0, 0)),
    out_shape=jax.ShapeDtypeStruct((T, NP, C), _f32),
)


BN_D = 1000


def _out_body(x_ref, w_ref, b_ref, out_ref):
    xb = x_ref[...]
    m = xb[0]
    for t in range(1, T):
        m = m + xb[t]
    m = m * (1.0 / T)
    out_ref[...] = jnp.dot(m, w_ref[...].T, preferred_element_type=_f32) \
        + b_ref[...]


_out_call = pl.pallas_call(
    _out_body,
    grid=(N // BN_D,),
    in_specs=[
        pl.BlockSpec((T, BN_D, C), lambda i: (0, i, 0)),
        pl.BlockSpec((C, C), lambda i: (0, 0)),
        pl.BlockSpec((1, C), lambda i: (0, 0)),
    ],
    out_specs=pl.BlockSpec((BN_D, C), lambda i: (i, 0)),
    out_shape=jax.ShapeDtypeStruct((N, C), _f32),
)


def _bucketize(row, col, w):
    bkt = col // RPT
    key = bkt * NSRC + (jnp.arange(E, dtype=_i32) // EPW)
    order = jnp.argsort(key)
    keys = key[order]
    counts = jnp.zeros((NB * NSRC,), _i32).at[keys].add(1)
    starts = jnp.concatenate(
        [jnp.zeros((1,), _i32), jnp.cumsum(counts)[:-1].astype(_i32)])
    rank = jnp.arange(E, dtype=_i32) - starts[keys]
    dest = keys * RCAP + rank
    brow = jnp.zeros((NB * NSRC * RCAP,), _i32).at[dest].set(row[order])
    bcl = jnp.zeros((NB * NSRC * RCAP,), _i32).at[dest].set(
        col[order] - (keys // NSRC) * RPT)
    bw = jnp.zeros((NB * NSRC * RCAP,), _f32).at[dest].set(w[order])
    return brow, bcl, bw, counts.reshape(NB, NSRC)


def kernel(x_seq, edge_index, edge_weight, l0_wc, l0_bc, l0_wg, l0_bg,
           l0_gamma, l0_beta, l1_wc, l1_bc, l1_wg, l1_bg, l1_gamma, l1_beta,
           out_w, out_b):
    row = edge_index[0]
    col = edge_index[1]
    w = edge_weight

    brow, bcl, bw, counts2 = _bucketize(row, col, w)
    degp = _deg_kernel(bcl, bw, counts2).reshape(2, NP, DL)

    x = jnp.zeros((T, NP, C), _f32).at[:, :N, :].set(x_seq)
    for (wc, bc, wg, gamma, beta) in (
            (l0_wc, l0_bc, l0_wg, l0_gamma, l0_beta),
            (l1_wc, l1_bc, l1_wg, l1_gamma, l1_beta)):
        wc_r = jnp.transpose(wc, (2, 0, 1))
        hpp = _mm_call(x, wc_r, wg, bc.reshape(1, C), degp)
        s = _spmm_kernel(hpp.reshape(T * NP, C), brow, bcl, bw, counts2)
        x = _bn_call(s.reshape(T, NP, C), hpp, degp,
                     gamma.reshape(1, C), beta.reshape(1, C))
    return _out_call(x[:, :N, :], out_w, out_b.reshape(1, C))


# speedup vs baseline: 1.7295x; 1.7295x over previous
import functools

import jax
import jax.numpy as jnp
from jax import lax
from jax.experimental import pallas as pl
from jax.experimental.pallas import tpu as pltpu
from jax.experimental.pallas import tpu_sc as plsc

N = 10000
E = 320000
T = 8
C = 128
NP = 10240
NC = 2
NS = 16
RPT = NP // NS
TPS = T // NC

NB = 16
NSRC = 32
EPW = E // NSRC
RCAP = 10272
SCH = 1024
K = 32
DL = 16

_f32 = jnp.float32
_i32 = jnp.int32

_sc_mesh = plsc.VectorSubcoreMesh(
    core_axis_name="c", subcore_axis_name="s", num_cores=NC, num_subcores=NS)


def _extract(v0, v1, j):
    return v0[j] if j < 16 else v1[j - 16]


def _dyn_lane(v0, v1, j):
    acc = v0[0]
    for k in range(1, 16):
        acc = jnp.where(j == k, v0[k], acc)
    for k in range(16):
        acc = jnp.where(j == k + 16, v1[k], acc)
    return acc


@functools.partial(
    pl.kernel,
    out_type=jax.ShapeDtypeStruct((NC * NP, DL), _f32),
    mesh=_sc_mesh,
    scratch_types=[
        pltpu.VMEM((NSRC,), _i32),
        pltpu.VMEM((SCH,), _i32),
        pltpu.VMEM((SCH,), _f32),
        pltpu.VMEM((RPT, DL), _f32),
    ],
)
def _deg_kernel(bcl, bw, counts2, out, cntv, scl, sw, dacc):
    cid = lax.axis_index("c")
    b = lax.axis_index("s")
    pltpu.sync_copy(counts2.at[b], cntv)
    cv0 = cntv[pl.ds(0, 16)]
    cv1 = cntv[pl.ds(16, 16)]

    zv = jnp.zeros((DL,), _f32)

    def zr(i, carry):
        dacc[i, :] = zv
        return carry

    lax.fori_loop(0, RPT, zr, 0)

    cvsel = jnp.where(cid == 0, cv0, cv1)

    def sloop(sl, carry):
        s = cid * (NSRC // NC) + sl
        cnt = _dyn_lane(cvsel, cvsel, sl)
        roff = (b * NSRC + s) * RCAP
        nstage = (cnt + (SCH - 1)) // SCH

        def stage(si, carry1):
            off = roff + si * SCH
            pltpu.sync_copy(bcl.at[pl.ds(off, SCH)], scl)
            pltpu.sync_copy(bw.at[pl.ds(off, SCH)], sw)
            rem = jnp.minimum(cnt - si * SCH, SCH)
            nin = (rem + (K - 1)) // K

            def chunk(ci, carry2):
                base = ci * K
                c0 = scl[pl.ds(base, 16)]
                c1 = scl[pl.ds(base + 16, 16)]
                w0 = sw[pl.ds(base, 16)]
                w1 = sw[pl.ds(base + 16, 16)]
                for j in range(K):
                    cl = _extract(c0, c1, j)
                    wj = _extract(w0, w1, j)
                    dacc[cl, :] = dacc[cl, :] + jnp.full((DL,), wj, _f32)
                return carry2

            lax.fori_loop(0, nin, chunk, 0)
            return carry1

        lax.fori_loop(0, nstage, stage, 0)
        return carry

    lax.fori_loop(0, NSRC // NC, sloop, 0)

    pltpu.sync_copy(dacc, out.at[pl.ds(cid * NP + b * RPT, RPT)])


NSLOT = 4
RTILE = RPT // NS


@functools.partial(
    pl.kernel,
    out_type=jax.ShapeDtypeStruct((T * NP, C), _f32),
    mesh=_sc_mesh,
    scratch_types=[
        pltpu.VMEM((NSRC,), _i32),
        pltpu.VMEM((SCH,), _i32),
        pltpu.VMEM((SCH,), _i32),
        pltpu.VMEM((SCH,), _f32),
        [pltpu.VMEM((K,), _i32) for _ in range(NSLOT)],
        [pltpu.VMEM((K,), _i32) for _ in range(NSLOT)],
        [pltpu.VMEM((K, C), _f32) for _ in range(NSLOT)],
        pltpu.VMEM((RTILE, C), _f32),
        pltpu.VMEM_SHARED((RPT, C), _f32),
        [pltpu.SemaphoreType.DMA for _ in range(NSLOT)],
        [pltpu.SemaphoreType.DMA for _ in range(NSLOT)],
    ],
)
def _spmm_kernel(hpp, brow, bcl, bw, counts2, out,
                 cntv, srow, scl, sw, idx, colb, gbuf, zbuf, acc,
                 semg, sems):
    cid = lax.axis_index("c")
    sid = lax.axis_index("s")

    zv = jnp.zeros((16,), _f32)

    def zb(i, carry):
        for v in range(C // 16):
            zbuf[i, pl.ds(v * 16, 16)] = zv
        return carry

    lax.fori_loop(0, RTILE, zb, 0)

    def build(k, ci, toff):
        base = ci * K
        idx[k][pl.ds(0, 16)] = srow[pl.ds(base, 16)] + toff
        idx[k][pl.ds(16, 16)] = srow[pl.ds(base + 16, 16)] + toff
        colb[k][pl.ds(0, 16)] = scl[pl.ds(base, 16)]
        colb[k][pl.ds(16, 16)] = scl[pl.ds(base + 16, 16)]
        pltpu.async_copy(hpp.at[idx[k]], gbuf[k], semg[k])

    def scale(k, ci):
        base = ci * K
        w0 = sw[pl.ds(base, 16)]
        w1 = sw[pl.ds(base + 16, 16)]
        for j in range(K):
            sv = jnp.full((16,), _extract(w0, w1, j), _f32)
            for v in range(C // 16):
                gbuf[k][j, pl.ds(v * 16, 16)] = \
                    gbuf[k][j, pl.ds(v * 16, 16)] * sv

    def tloop(tl, tcarry):
        t = cid * TPS + tl
        toff = t * NP

        def bloop(b, bcarry):
            pltpu.sync_copy(counts2.at[b], cntv)
            cv0 = cntv[pl.ds(0, 16)]
            cv1 = cntv[pl.ds(16, 16)]
            pltpu.sync_copy(zbuf, acc.at[pl.ds(sid * RTILE, RTILE)])
            plsc.subcore_barrier()

            for sreg in range(2):
                s = sid * 2 + sreg
                cnt = _dyn_lane(cv0, cv1, s)
                roff = (b * NSRC + s) * RCAP
                nstage = (cnt + (SCH - 1)) // SCH

                def stage(si, carry1):
                    off = roff + si * SCH
                    pltpu.sync_copy(brow.at[pl.ds(off, SCH)], srow)
                    pltpu.sync_copy(bcl.at[pl.ds(off, SCH)], scl)
                    pltpu.sync_copy(bw.at[pl.ds(off, SCH)], sw)
                    rem = jnp.minimum(cnt - si * SCH, SCH)
                    nin = (rem + (K - 1)) // K

                    for k in range(2):
                        @pl.when(k < nin)
                        def _prime(k=k):
                            build(k, k, toff)

                    def quad(qi, carry2):
                        for k in range(NSLOT):
                            ci = qi * NSLOT + k

                            @pl.when(ci < nin)
                            def _do(k=k, ci=ci):
                                pltpu.make_async_copy(
                                    hpp.at[idx[k]], gbuf[k], semg[k]).wait()
                                scale(k, ci)
                                pltpu.async_copy(
                                    gbuf[k], acc.at[colb[k]], sems[k],
                                    add=True)
                                m = (k + 2) % NSLOT

                                @pl.when(ci + 2 < nin)
                                def _prep(k=k, ci=ci, m=m):
                                    @pl.when(ci + 2 >= NSLOT)
                                    def _drain(m=m):
                                        pltpu.make_async_copy(
                                            gbuf[m], acc.at[colb[m]],
                                            sems[m]).wait()
                                    build(m, ci + 2, toff)
                        return carry2

                    lax.fori_loop(0, (nin + NSLOT - 1) // NSLOT, quad, 0)
                    for k in range(NSLOT):
                        @pl.when(k < nin)
                        def _fdrain(k=k):
                            pltpu.make_async_copy(
                                gbuf[k], acc.at[colb[k]], sems[k]).wait()
                    return carry1

                lax.fori_loop(0, nstage, stage, 0)

            plsc.subcore_barrier()
            pltpu.sync_copy(
                acc.at[pl.ds(sid * RTILE, RTILE)],
                out.at[pl.ds(toff + b * RPT + sid * RTILE, RTILE)])
            return bcarry

        lax.fori_loop(0, NB, bloop, 0)
        return tcarry

    lax.fori_loop(0, TPS, tloop, 0)


BN_A = 1024


def _mm_body(x_ref, wc_ref, wg_ref, bc_ref, degp_ref, out_ref):
    xb = x_ref[...]
    wg_t = wg_ref[...].T
    m = [jnp.dot(wc_ref[k].T, wg_t, preferred_element_type=_f32)
         for k in range(3)]
    hb = jnp.dot(bc_ref[...], wg_t, preferred_element_type=_f32)
    pb = degp_ref[...]
    dis = lax.rsqrt(1.0 + pb[0, :, 0:1] + pb[1, :, 0:1])
    for t in range(T):
        acc = jnp.dot(xb[t], m[1], preferred_element_type=_f32) + hb
        if t > 0:
            acc = acc + jnp.dot(xb[t - 1], m[0], preferred_element_type=_f32)
        if t < T - 1:
            acc = acc + jnp.dot(xb[t + 1], m[2], preferred_element_type=_f32)
        out_ref[t] = acc * dis


_mm_call = pl.pallas_call(
    _mm_body,
    grid=(NP // BN_A,),
    in_specs=[
        pl.BlockSpec((T, BN_A, C), lambda i: (0, i, 0)),
        pl.BlockSpec((3, C, C), lambda i: (0, 0, 0)),
        pl.BlockSpec((C, C), lambda i: (0, 0)),
        pl.BlockSpec((1, C), lambda i: (0, 0)),
        pl.BlockSpec((2, BN_A, DL), lambda i: (0, i, 0)),
    ],
    out_specs=pl.BlockSpec((T, BN_A, C), lambda i: (0, i, 0)),
    out_shape=jax.ShapeDtypeStruct((T, NP, C), _f32),
)


def _bn_body(s_ref, h_ref, degp_ref, gamma_ref, beta_ref, out_ref):
    sb = s_ref[0]
    hb = h_ref[0]
    pb = degp_ref[...]
    dis = lax.rsqrt(1.0 + pb[0, :, 0:1] + pb[1, :, 0:1])
    o = dis * (sb + hb)
    mask = lax.broadcasted_iota(_i32, (NP, 1), 0) < N
    om = jnp.where(mask, o, 0.0)
    mu = jnp.sum(om, axis=0, keepdims=True) * (1.0 / N)
    d = jnp.where(mask, o - mu, 0.0)
    var = jnp.sum(d * d, axis=0, keepdims=True) * (1.0 / N)
    scale = gamma_ref[...] * lax.rsqrt(var + 1e-5)
    out_ref[0] = jnp.maximum((o - mu) * scale + beta_ref[...], 0.0)


_bn_call = pl.pallas_call(
    _bn_body,
    grid=(T,),
    in_specs=[
        pl.BlockSpec((1, NP, C), lambda t: (t, 0, 0)),
        pl.BlockSpec((1, NP, C), lambda t: (t, 0, 0)),
        pl.BlockSpec((2, NP, DL), lambda t: (0, 0, 0)),
        pl.BlockSpec((1, C), lambda t: (0, 0)),
        pl.BlockSpec((1, C), lambda t: (0, 0)),
    ],
    out_specs=pl.BlockSpec((1, NP, C), lambda t: (t, 0, 0)),
    out_shape=jax.ShapeDtypeStruct((T, NP, C), _f32),
)


BN_D = 1000


def _out_body(x_ref, w_ref, b_ref, out_ref):
    xb = x_ref[...]
    m = xb[0]
    for t in range(1, T):
        m = m + xb[t]
    m = m * (1.0 / T)
    out_ref[...] = jnp.dot(m, w_ref[...].T, preferred_element_type=_f32) \
        + b_ref[...]


_out_call = pl.pallas_call(
    _out_body,
    grid=(N // BN_D,),
    in_specs=[
        pl.BlockSpec((T, BN_D, C), lambda i: (0, i, 0)),
        pl.BlockSpec((C, C), lambda i: (0, 0)),
        pl.BlockSpec((1, C), lambda i: (0, 0)),
    ],
    out_specs=pl.BlockSpec((BN_D, C), lambda i: (i, 0)),
    out_shape=jax.ShapeDtypeStruct((N, C), _f32),
)


def _bucketize(row, col, w):
    bkt = col // RPT
    key = bkt * NSRC + (jnp.arange(E, dtype=_i32) // EPW)
    order = jnp.argsort(key)
    keys = key[order]
    counts = jnp.zeros((NB * NSRC,), _i32).at[keys].add(1)
    starts = jnp.concatenate(
        [jnp.zeros((1,), _i32), jnp.cumsum(counts)[:-1].astype(_i32)])
    rank = jnp.arange(E, dtype=_i32) - starts[keys]
    dest = keys * RCAP + rank
    brow = jnp.zeros((NB * NSRC * RCAP,), _i32).at[dest].set(row[order])
    bcl = jnp.zeros((NB * NSRC * RCAP,), _i32).at[dest].set(
        col[order] - (keys // NSRC) * RPT)
    bw = jnp.zeros((NB * NSRC * RCAP,), _f32).at[dest].set(w[order])
    return brow, bcl, bw, counts.reshape(NB, NSRC)


def kernel(x_seq, edge_index, edge_weight, l0_wc, l0_bc, l0_wg, l0_bg,
           l0_gamma, l0_beta, l1_wc, l1_bc, l1_wg, l1_bg, l1_gamma, l1_beta,
           out_w, out_b):
    row = edge_index[0]
    col = edge_index[1]
    w = edge_weight

    brow, bcl, bw, counts2 = _bucketize(row, col, w)
    degp = _deg_kernel(bcl, bw, counts2).reshape(2, NP, DL)

    x = jnp.zeros((T, NP, C), _f32).at[:, :N, :].set(x_seq)
    for (wc, bc, wg, gamma, beta) in (
            (l0_wc, l0_bc, l0_wg, l0_gamma, l0_beta),
            (l1_wc, l1_bc, l1_wg, l1_gamma, l1_beta)):
        wc_r = jnp.transpose(wc, (2, 0, 1))
        hpp = _mm_call(x, wc_r, wg, bc.reshape(1, C), degp)
        s = _spmm_kernel(hpp.reshape(T * NP, C), brow, bcl, bw, counts2)
        x = _bn_call(s.reshape(T, NP, C), hpp, degp,
                     gamma.reshape(1, C), beta.reshape(1, C))
    return _out_call(x[:, :N, :], out_w, out_b.reshape(1, C))


# speedup vs baseline: 3.3268x; 1.9235x over previous
import functools

import jax
import jax.numpy as jnp
from jax import lax
from jax.experimental import pallas as pl
from jax.experimental.pallas import tpu as pltpu
from jax.experimental.pallas import tpu_sc as plsc

N = 10000
E = 320000
T = 8
C = 128
NP = 10240
NC = 2
NS = 16
RPT = NP // NS
TPS = T // NC

NB = 16
NSRC = 32
EPW = E // NSRC
RCAP = 10272
SCH = 1024
K = 32
DL = 16

_f32 = jnp.float32
_i32 = jnp.int32

_sc_mesh = plsc.VectorSubcoreMesh(
    core_axis_name="c", subcore_axis_name="s", num_cores=NC, num_subcores=NS)


def _extract(v0, v1, j):
    return v0[j] if j < 16 else v1[j - 16]


def _dyn_lane(v0, v1, j):
    acc = v0[0]
    for k in range(1, 16):
        acc = jnp.where(j == k, v0[k], acc)
    for k in range(16):
        acc = jnp.where(j == k + 16, v1[k], acc)
    return acc


@functools.partial(
    pl.kernel,
    out_type=(
        jax.ShapeDtypeStruct((NB * NSRC * RCAP,), _i32),
        jax.ShapeDtypeStruct((NB * NSRC * RCAP,), _i32),
        jax.ShapeDtypeStruct((NB * NSRC * RCAP,), _f32),
        jax.ShapeDtypeStruct((NSRC * NB,), _i32),
    ),
    mesh=_sc_mesh,
    compiler_params=pltpu.CompilerParams(needs_layout_passes=False),
    scratch_types=[
        pltpu.VMEM((EPW,), _i32),
        pltpu.VMEM((EPW,), _i32),
        pltpu.VMEM((EPW,), _f32),
        pltpu.VMEM((RCAP,), _i32),
        pltpu.VMEM((RCAP,), _i32),
        pltpu.VMEM((RCAP,), _f32),
        pltpu.VMEM((16,), _i32),
    ],
)
def _scan_kernel(rowv, colv, wv, brow, bcl, bw, counts,
                 srowS, scolS, swS, obrow, obcl, obw, cbuf):
    cid = lax.axis_index("c")
    sid = lax.axis_index("s")
    wid = cid * NS + sid
    pltpu.sync_copy(rowv.at[pl.ds(wid * EPW, EPW)], srowS)
    pltpu.sync_copy(colv.at[pl.ds(wid * EPW, EPW)], scolS)
    pltpu.sync_copy(wv.at[pl.ds(wid * EPW, EPW)], swS)

    zvi = jnp.zeros((16,), _i32)
    zvf = jnp.zeros((16,), _f32)
    lanes = lax.broadcasted_iota(_i32, (16,), 0)

    def bloop(b, countsv):
        lo = b * RPT

        def chunk(ci, cursor):
            base = ci * 16
            c = scolS[pl.ds(base, 16)]
            r = srowS[pl.ds(base, 16)]
            wch = swS[pl.ds(base, 16)]
            m = (c >= lo) & (c < lo + RPT)
            incl = plsc.cumsum(m.astype(_i32))
            dest = jnp.where(m, cursor + incl - 1, (RCAP - 16) + lanes)
            plsc.store_scatter(obrow, [dest], r)
            plsc.store_scatter(obcl, [dest], c - lo)
            plsc.store_scatter(obw, [dest], wch)
            return cursor + incl[15]

        cursor = lax.fori_loop(0, EPW // 16, chunk, jnp.int32(0))
        obrow[pl.ds(cursor, 16)] = zvi
        obrow[pl.ds(cursor + 16, 16)] = zvi
        obcl[pl.ds(cursor, 16)] = zvi
        obcl[pl.ds(cursor + 16, 16)] = zvi
        obw[pl.ds(cursor, 16)] = zvf
        obw[pl.ds(cursor + 16, 16)] = zvf
        roff = (b * NSRC + wid) * RCAP
        pltpu.sync_copy(obrow, brow.at[pl.ds(roff, RCAP)])
        pltpu.sync_copy(obcl, bcl.at[pl.ds(roff, RCAP)])
        pltpu.sync_copy(obw, bw.at[pl.ds(roff, RCAP)])
        return jnp.where(lanes == b, cursor, countsv)

    countsv = lax.fori_loop(0, NB, bloop, jnp.zeros((16,), _i32))
    cbuf[:] = countsv
    pltpu.sync_copy(cbuf, counts.at[pl.ds(wid * 16, 16)])


@functools.partial(
    pl.kernel,
    out_type=jax.ShapeDtypeStruct((NC * NP, DL), _f32),
    mesh=_sc_mesh,
    scratch_types=[
        pltpu.VMEM((NSRC * NB,), _i32),
        pltpu.VMEM((SCH,), _i32),
        pltpu.VMEM((SCH,), _f32),
        pltpu.VMEM((RPT, DL), _f32),
    ],
)
def _deg_kernel(bcl, bw, counts2, out, cntv, scl, sw, dacc):
    cid = lax.axis_index("c")
    b = lax.axis_index("s")
    pltpu.sync_copy(counts2, cntv)

    zv = jnp.zeros((DL,), _f32)

    def zr(i, carry):
        dacc[i, :] = zv
        return carry

    lax.fori_loop(0, RPT, zr, 0)

    def sloop(sl, carry):
        s = cid * (NSRC // NC) + sl
        cvec = cntv[pl.ds(s * 16, 16)]
        cnt = _dyn_lane(cvec, cvec, b)
        roff = (b * NSRC + s) * RCAP
        nstage = (cnt + (SCH - 1)) // SCH

        def stage(si, carry1):
            off = roff + si * SCH
            pltpu.sync_copy(bcl.at[pl.ds(off, SCH)], scl)
            pltpu.sync_copy(bw.at[pl.ds(off, SCH)], sw)
            rem = jnp.minimum(cnt - si * SCH, SCH)
            nin = (rem + (K - 1)) // K

            def chunk(ci, carry2):
                base = ci * K
                c0 = scl[pl.ds(base, 16)]
                c1 = scl[pl.ds(base + 16, 16)]
                w0 = sw[pl.ds(base, 16)]
                w1 = sw[pl.ds(base + 16, 16)]
                for j in range(K):
                    cl = _extract(c0, c1, j)
                    wj = _extract(w0, w1, j)
                    dacc[cl, :] = dacc[cl, :] + jnp.full((DL,), wj, _f32)
                return carry2

            lax.fori_loop(0, nin, chunk, 0)
            return carry1

        lax.fori_loop(0, nstage, stage, 0)
        return carry

    lax.fori_loop(0, NSRC // NC, sloop, 0)

    pltpu.sync_copy(dacc, out.at[pl.ds(cid * NP + b * RPT, RPT)])


NSLOT = 4
RTILE = RPT // NS


@functools.partial(
    pl.kernel,
    out_type=jax.ShapeDtypeStruct((T * NP, C), _f32),
    mesh=_sc_mesh,
    scratch_types=[
        pltpu.VMEM((NSRC * NB,), _i32),
        pltpu.VMEM((SCH,), _i32),
        pltpu.VMEM((SCH,), _i32),
        pltpu.VMEM((SCH,), _f32),
        [pltpu.VMEM((K,), _i32) for _ in range(NSLOT)],
        [pltpu.VMEM((K,), _i32) for _ in range(NSLOT)],
        [pltpu.VMEM((K, C), _f32) for _ in range(NSLOT)],
        pltpu.VMEM((RTILE, C), _f32),
        pltpu.VMEM_SHARED((RPT, C), _f32),
        [pltpu.SemaphoreType.DMA for _ in range(NSLOT)],
        [pltpu.SemaphoreType.DMA for _ in range(NSLOT)],
    ],
)
def _spmm_kernel(hpp, brow, bcl, bw, counts2, out,
                 cntv, srow, scl, sw, idx, colb, gbuf, zbuf, acc,
                 semg, sems):
    cid = lax.axis_index("c")
    sid = lax.axis_index("s")
    pltpu.sync_copy(counts2, cntv)

    zv = jnp.zeros((16,), _f32)

    def zb(i, carry):
        for v in range(C // 16):
            zbuf[i, pl.ds(v * 16, 16)] = zv
        return carry

    lax.fori_loop(0, RTILE, zb, 0)

    def build(k, ci, toff):
        base = ci * K
        idx[k][pl.ds(0, 16)] = srow[pl.ds(base, 16)] + toff
        idx[k][pl.ds(16, 16)] = srow[pl.ds(base + 16, 16)] + toff
        colb[k][pl.ds(0, 16)] = scl[pl.ds(base, 16)]
        colb[k][pl.ds(16, 16)] = scl[pl.ds(base + 16, 16)]
        pltpu.async_copy(hpp.at[idx[k]], gbuf[k], semg[k])

    def scale(k, ci):
        base = ci * K
        w0 = sw[pl.ds(base, 16)]
        w1 = sw[pl.ds(base + 16, 16)]
        for j in range(K):
            sv = jnp.full((16,), _extract(w0, w1, j), _f32)
            for v in range(C // 16):
                gbuf[k][j, pl.ds(v * 16, 16)] = \
                    gbuf[k][j, pl.ds(v * 16, 16)] * sv

    def tloop(tl, tcarry):
        t = cid * TPS + tl
        toff = t * NP

        def bloop(b, bcarry):
            pltpu.sync_copy(zbuf, acc.at[pl.ds(sid * RTILE, RTILE)])
            plsc.subcore_barrier()

            for sreg in range(2):
                s = sid * 2 + sreg
                cvec = cntv[pl.ds(s * 16, 16)]
                cnt = _dyn_lane(cvec, cvec, b)
                roff = (b * NSRC + s) * RCAP
                nstage = (cnt + (SCH - 1)) // SCH

                def stage(si, carry1):
                    off = roff + si * SCH
                    pltpu.sync_copy(brow.at[pl.ds(off, SCH)], srow)
                    pltpu.sync_copy(bcl.at[pl.ds(off, SCH)], scl)
                    pltpu.sync_copy(bw.at[pl.ds(off, SCH)], sw)
                    rem = jnp.minimum(cnt - si * SCH, SCH)
                    nin = (rem + (K - 1)) // K

                    for k in range(2):
                        @pl.when(k < nin)
                        def _prime(k=k):
                            build(k, k, toff)

                    def quad(qi, carry2):
                        for k in range(NSLOT):
                            ci = qi * NSLOT + k

                            @pl.when(ci < nin)
                            def _do(k=k, ci=ci):
                                pltpu.make_async_copy(
                                    hpp.at[idx[k]], gbuf[k], semg[k]).wait()
                                scale(k, ci)
                                pltpu.async_copy(
                                    gbuf[k], acc.at[colb[k]], sems[k],
                                    add=True)
                                m = (k + 2) % NSLOT

                                @pl.when(ci + 2 < nin)
                                def _prep(k=k, ci=ci, m=m):
                                    @pl.when(ci + 2 >= NSLOT)
                                    def _drain(m=m):
                                        pltpu.make_async_copy(
                                            gbuf[m], acc.at[colb[m]],
                                            sems[m]).wait()
                                    build(m, ci + 2, toff)
                        return carry2

                    lax.fori_loop(0, (nin + NSLOT - 1) // NSLOT, quad, 0)
                    for k in range(NSLOT):
                        @pl.when(k < nin)
                        def _fdrain(k=k):
                            pltpu.make_async_copy(
                                gbuf[k], acc.at[colb[k]], sems[k]).wait()
                    return carry1

                lax.fori_loop(0, nstage, stage, 0)

            plsc.subcore_barrier()
            pltpu.sync_copy(
                acc.at[pl.ds(sid * RTILE, RTILE)],
                out.at[pl.ds(toff + b * RPT + sid * RTILE, RTILE)])
            return bcarry

        lax.fori_loop(0, NB, bloop, 0)
        return tcarry

    lax.fori_loop(0, TPS, tloop, 0)


BN_A = 1024


def _mm_body(x_ref, wc_ref, wg_ref, bc_ref, degp_ref, out_ref):
    xb = x_ref[...]
    wg_t = wg_ref[...].T
    m = [jnp.dot(wc_ref[k].T, wg_t, preferred_element_type=_f32)
         for k in range(3)]
    hb = jnp.dot(bc_ref[...], wg_t, preferred_element_type=_f32)
    pb = degp_ref[...]
    dis = lax.rsqrt(1.0 + pb[0, :, 0:1] + pb[1, :, 0:1])
    for t in range(T):
        acc = jnp.dot(xb[t], m[1], preferred_element_type=_f32) + hb
        if t > 0:
            acc = acc + jnp.dot(xb[t - 1], m[0], preferred_element_type=_f32)
        if t < T - 1:
            acc = acc + jnp.dot(xb[t + 1], m[2], preferred_element_type=_f32)
        out_ref[t] = acc * dis


_mm_call = pl.pallas_call(
    _mm_body,
    grid=(NP // BN_A,),
    in_specs=[
        pl.BlockSpec((T, BN_A, C), lambda i: (0, i, 0)),
        pl.BlockSpec((3, C, C), lambda i: (0, 0, 0)),
        pl.BlockSpec((C, C), lambda i: (0, 0)),
        pl.BlockSpec((1, C), lambda i: (0, 0)),
        pl.BlockSpec((2, BN_A, DL), lambda i: (0, i, 0)),
    ],
    out_specs=pl.BlockSpec((T, BN_A, C), lambda i: (0, i, 0)),
    out_shape=jax.ShapeDtypeStruct((T, NP, C), _f32),
)


def _bn_body(s_ref, h_ref, degp_ref, gamma_ref, beta_ref, out_ref):
    sb = s_ref[0]
    hb = h_ref[0]
    pb = degp_ref[...]
    dis = lax.rsqrt(1.0 + pb[0, :, 0:1] + pb[1, :, 0:1])
    o = dis * (sb + hb)
    mask = lax.broadcasted_iota(_i32, (NP, 1), 0) < N
    om = jnp.where(mask, o, 0.0)
    mu = jnp.sum(om, axis=0, keepdims=True) * (1.0 / N)
    d = jnp.where(mask, o - mu, 0.0)
    var = jnp.sum(d * d, axis=0, keepdims=True) * (1.0 / N)
    scale = gamma_ref[...] * lax.rsqrt(var + 1e-5)
    out_ref[0] = jnp.maximum((o - mu) * scale + beta_ref[...], 0.0)


_bn_call = pl.pallas_call(
    _bn_body,
    grid=(T,),
    in_specs=[
        pl.BlockSpec((1, NP, C), lambda t: (t, 0, 0)),
        pl.BlockSpec((1, NP, C), lambda t: (t, 0, 0)),
        pl.BlockSpec((2, NP, DL), lambda t: (0, 0, 0)),
        pl.BlockSpec((1, C), lambda t: (0, 0)),
        pl.BlockSpec((1, C), lambda t: (0, 0)),
    ],
    out_specs=pl.BlockSpec((1, NP, C), lambda t: (t, 0, 0)),
    out_shape=jax.ShapeDtypeStruct((T, NP, C), _f32),
)


BN_D = 1000


def _out_body(x_ref, w_ref, b_ref, out_ref):
    xb = x_ref[...]
    m = xb[0]
    for t in range(1, T):
        m = m + xb[t]
    m = m * (1.0 / T)
    out_ref[...] = jnp.dot(m, w_ref[...].T, preferred_element_type=_f32) \
        + b_ref[...]


_out_call = pl.pallas_call(
    _out_body,
    grid=(N // BN_D,),
    in_specs=[
        pl.BlockSpec((T, BN_D, C), lambda i: (0, i, 0)),
        pl.BlockSpec((C, C), lambda i: (0, 0)),
        pl.BlockSpec((1, C), lambda i: (0, 0)),
    ],
    out_specs=pl.BlockSpec((BN_D, C), lambda i: (i, 0)),
    out_shape=jax.ShapeDtypeStruct((N, C), _f32),
)


def kernel(x_seq, edge_index, edge_weight, l0_wc, l0_bc, l0_wg, l0_bg,
           l0_gamma, l0_beta, l1_wc, l1_bc, l1_wg, l1_bg, l1_gamma, l1_beta,
           out_w, out_b):
    row = edge_index[0]
    col = edge_index[1]
    w = edge_weight

    brow, bcl, bw, counts2 = _scan_kernel(row, col, w)
    degp = _deg_kernel(bcl, bw, counts2).reshape(2, NP, DL)

    x = jnp.zeros((T, NP, C), _f32).at[:, :N, :].set(x_seq)
    for (wc, bc, wg, gamma, beta) in (
            (l0_wc, l0_bc, l0_wg, l0_gamma, l0_beta),
            (l1_wc, l1_bc, l1_wg, l1_gamma, l1_beta)):
        wc_r = jnp.transpose(wc, (2, 0, 1))
        hpp = _mm_call(x, wc_r, wg, bc.reshape(1, C), degp)
        s = _spmm_kernel(hpp.reshape(T * NP, C), brow, bcl, bw, counts2)
        x = _bn_call(s.reshape(T, NP, C), hpp, degp,
                     gamma.reshape(1, C), beta.reshape(1, C))
    return _out_call(x[:, :N, :], out_w, out_b.reshape(1, C))


# speedup vs baseline: 3.5127x; 1.0559x over previous
import functools

import jax
import jax.numpy as jnp
from jax import lax
from jax.experimental import pallas as pl
from jax.experimental.pallas import tpu as pltpu
from jax.experimental.pallas import tpu_sc as plsc

N = 10000
E = 320000
T = 8
C = 128
NP = 10240
NC = 2
NS = 16
RPT = NP // NS
TPS = T // NC

NB = 16
NSRC = 32
EPW = E // NSRC
RCAP = 10272
SCH = 2048
K = 32
KD = 32
DL = 16

_f32 = jnp.float32
_i32 = jnp.int32

_sc_mesh = plsc.VectorSubcoreMesh(
    core_axis_name="c", subcore_axis_name="s", num_cores=NC, num_subcores=NS)


def _extract(v0, v1, j):
    return v0[j] if j < 16 else v1[j - 16]


def _dyn_lane(v0, v1, j):
    acc = v0[0]
    for k in range(1, 16):
        acc = jnp.where(j == k, v0[k], acc)
    for k in range(16):
        acc = jnp.where(j == k + 16, v1[k], acc)
    return acc


@functools.partial(
    pl.kernel,
    out_type=(
        jax.ShapeDtypeStruct((NB * NSRC * RCAP,), _i32),
        jax.ShapeDtypeStruct((NB * NSRC * RCAP,), _i32),
        jax.ShapeDtypeStruct((NB * NSRC * RCAP,), _f32),
        jax.ShapeDtypeStruct((NSRC * NB,), _i32),
    ),
    mesh=_sc_mesh,
    compiler_params=pltpu.CompilerParams(needs_layout_passes=False),
    scratch_types=[
        pltpu.VMEM((EPW,), _i32),
        pltpu.VMEM((EPW,), _i32),
        pltpu.VMEM((EPW,), _f32),
        pltpu.VMEM((RCAP,), _i32),
        pltpu.VMEM((RCAP,), _i32),
        pltpu.VMEM((RCAP,), _f32),
        pltpu.VMEM((16,), _i32),
    ],
)
def _scan_kernel(rowv, colv, wv, brow, bcl, bw, counts,
                 srowS, scolS, swS, obrow, obcl, obw, cbuf):
    cid = lax.axis_index("c")
    sid = lax.axis_index("s")
    wid = cid * NS + sid
    pltpu.sync_copy(rowv.at[pl.ds(wid * EPW, EPW)], srowS)
    pltpu.sync_copy(colv.at[pl.ds(wid * EPW, EPW)], scolS)
    pltpu.sync_copy(wv.at[pl.ds(wid * EPW, EPW)], swS)

    zvi = jnp.zeros((16,), _i32)
    zvf = jnp.zeros((16,), _f32)
    lanes = lax.broadcasted_iota(_i32, (16,), 0)

    def bloop(b, countsv):
        lo = b * RPT

        def chunk(ci, cursor):
            base = ci * 16
            c = scolS[pl.ds(base, 16)]
            r = srowS[pl.ds(base, 16)]
            wch = swS[pl.ds(base, 16)]
            m = (c >= lo) & (c < lo + RPT)
            incl = plsc.cumsum(m.astype(_i32))
            dest = jnp.where(m, cursor + incl - 1, (RCAP - 16) + lanes)
            plsc.store_scatter(obrow, [dest], r)
            plsc.store_scatter(obcl, [dest], c - lo)
            plsc.store_scatter(obw, [dest], wch)
            return cursor + incl[15]

        cursor = lax.fori_loop(0, EPW // 16, chunk, jnp.int32(0))
        obrow[pl.ds(cursor, 16)] = zvi
        obrow[pl.ds(cursor + 16, 16)] = zvi
        obcl[pl.ds(cursor, 16)] = zvi
        obcl[pl.ds(cursor + 16, 16)] = zvi
        obw[pl.ds(cursor, 16)] = zvf
        obw[pl.ds(cursor + 16, 16)] = zvf
        roff = (b * NSRC + wid) * RCAP
        pltpu.sync_copy(obrow, brow.at[pl.ds(roff, RCAP)])
        pltpu.sync_copy(obcl, bcl.at[pl.ds(roff, RCAP)])
        pltpu.sync_copy(obw, bw.at[pl.ds(roff, RCAP)])
        return jnp.where(lanes == b, cursor, countsv)

    countsv = lax.fori_loop(0, NB, bloop, jnp.zeros((16,), _i32))
    cbuf[:] = countsv
    pltpu.sync_copy(cbuf, counts.at[pl.ds(wid * 16, 16)])


@functools.partial(
    pl.kernel,
    out_type=jax.ShapeDtypeStruct((NC * NP, DL), _f32),
    mesh=_sc_mesh,
    scratch_types=[
        pltpu.VMEM((NSRC * NB,), _i32),
        pltpu.VMEM((SCH,), _i32),
        pltpu.VMEM((SCH,), _f32),
        pltpu.VMEM((RPT, DL), _f32),
    ],
)
def _deg_kernel(bcl, bw, counts2, out, cntv, scl, sw, dacc):
    cid = lax.axis_index("c")
    b = lax.axis_index("s")
    pltpu.sync_copy(counts2, cntv)

    zv = jnp.zeros((DL,), _f32)

    def zr(i, carry):
        dacc[i, :] = zv
        return carry

    lax.fori_loop(0, RPT, zr, 0)

    def sloop(sl, carry):
        s = cid * (NSRC // NC) + sl
        cvec = cntv[pl.ds(s * 16, 16)]
        cnt = _dyn_lane(cvec, cvec, b)
        roff = (b * NSRC + s) * RCAP
        nstage = (cnt + (SCH - 1)) // SCH

        def stage(si, carry1):
            off = roff + si * SCH
            pltpu.sync_copy(bcl.at[pl.ds(off, SCH)], scl)
            pltpu.sync_copy(bw.at[pl.ds(off, SCH)], sw)
            rem = jnp.minimum(cnt - si * SCH, SCH)
            nin = (rem + (KD - 1)) // KD

            def chunk(ci, carry2):
                base = ci * KD
                c0 = scl[pl.ds(base, 16)]
                c1 = scl[pl.ds(base + 16, 16)]
                w0 = sw[pl.ds(base, 16)]
                w1 = sw[pl.ds(base + 16, 16)]
                for j in range(KD):
                    cl = _extract(c0, c1, j)
                    wj = _extract(w0, w1, j)
                    dacc[cl, :] = dacc[cl, :] + jnp.full((DL,), wj, _f32)
                return carry2

            lax.fori_loop(0, nin, chunk, 0)
            return carry1

        lax.fori_loop(0, nstage, stage, 0)
        return carry

    lax.fori_loop(0, NSRC // NC, sloop, 0)

    pltpu.sync_copy(dacc, out.at[pl.ds(cid * NP + b * RPT, RPT)])


NSLOT = 6
LEAD = NSLOT // 2
RTILE = RPT // NS


@functools.partial(
    pl.kernel,
    out_type=jax.ShapeDtypeStruct((T * NP, C), _f32),
    mesh=_sc_mesh,
    scratch_types=[
        pltpu.VMEM((NSRC * NB,), _i32),
        pltpu.VMEM((SCH,), _i32),
        pltpu.VMEM((SCH,), _i32),
        pltpu.VMEM((SCH,), _f32),
        [pltpu.VMEM((K,), _i32) for _ in range(NSLOT)],
        [pltpu.VMEM((K,), _i32) for _ in range(NSLOT)],
        [pltpu.VMEM((K, C), _f32) for _ in range(NSLOT)],
        pltpu.VMEM((RTILE, C), _f32),
        pltpu.VMEM_SHARED((RPT, C), _f32),
        [pltpu.SemaphoreType.DMA for _ in range(NSLOT)],
        [pltpu.SemaphoreType.DMA for _ in range(NSLOT)],
    ],
)
def _spmm_kernel(hpp, brow, bcl, bw, counts2, out,
                 cntv, srow, scl, sw, idx, colb, gbuf, zbuf, acc,
                 semg, sems):
    cid = lax.axis_index("c")
    sid = lax.axis_index("s")
    pltpu.sync_copy(counts2, cntv)

    zv = jnp.zeros((16,), _f32)

    def zb(i, carry):
        for v in range(C // 16):
            zbuf[i, pl.ds(v * 16, 16)] = zv
        return carry

    lax.fori_loop(0, RTILE, zb, 0)

    def build(k, ci, toff):
        base = ci * K
        for q in range(K // 16):
            idx[k][pl.ds(q * 16, 16)] = srow[pl.ds(base + q * 16, 16)] + toff
            colb[k][pl.ds(q * 16, 16)] = scl[pl.ds(base + q * 16, 16)]
        pltpu.async_copy(hpp.at[idx[k]], gbuf[k], semg[k])

    def scale(k, ci):
        base = ci * K
        wvecs = [sw[pl.ds(base + q * 16, 16)] for q in range(K // 16)]
        for j in range(K):
            sv = jnp.full((16,), wvecs[j // 16][j % 16], _f32)
            for v in range(C // 16):
                gbuf[k][j, pl.ds(v * 16, 16)] = \
                    gbuf[k][j, pl.ds(v * 16, 16)] * sv

    def tloop(tl, tcarry):
        t = cid * TPS + tl
        toff = t * NP

        def bloop(b, bcarry):
            pltpu.sync_copy(zbuf, acc.at[pl.ds(sid * RTILE, RTILE)])
            plsc.subcore_barrier()

            for sreg in range(2):
                s = sid * 2 + sreg
                cvec = cntv[pl.ds(s * 16, 16)]
                cnt = _dyn_lane(cvec, cvec, b)
                roff = (b * NSRC + s) * RCAP
                nstage = (cnt + (SCH - 1)) // SCH

                def stage(si, carry1):
                    off = roff + si * SCH
                    pltpu.sync_copy(brow.at[pl.ds(off, SCH)], srow)
                    pltpu.sync_copy(bcl.at[pl.ds(off, SCH)], scl)
                    pltpu.sync_copy(bw.at[pl.ds(off, SCH)], sw)
                    rem = jnp.minimum(cnt - si * SCH, SCH)
                    nin = (rem + (K - 1)) // K

                    for k in range(LEAD):
                        @pl.when(k < nin)
                        def _prime(k=k):
                            build(k, k, toff)

                    def quad(qi, carry2):
                        for k in range(NSLOT):
                            ci = qi * NSLOT + k

                            @pl.when(ci < nin)
                            def _do(k=k, ci=ci):
                                pltpu.make_async_copy(
                                    hpp.at[idx[k]], gbuf[k], semg[k]).wait()
                                scale(k, ci)
                                pltpu.async_copy(
                                    gbuf[k], acc.at[colb[k]], sems[k],
                                    add=True)
                                m = (k + LEAD) % NSLOT

                                @pl.when(ci + LEAD < nin)
                                def _prep(k=k, ci=ci, m=m):
                                    @pl.when(ci + LEAD >= NSLOT)
                                    def _drain(m=m):
                                        pltpu.make_async_copy(
                                            gbuf[m], acc.at[colb[m]],
                                            sems[m]).wait()
                                    build(m, ci + LEAD, toff)
                        return carry2

                    lax.fori_loop(0, (nin + NSLOT - 1) // NSLOT, quad, 0)
                    for k in range(NSLOT):
                        @pl.when(k < nin)
                        def _fdrain(k=k):
                            pltpu.make_async_copy(
                                gbuf[k], acc.at[colb[k]], sems[k]).wait()
                    return carry1

                lax.fori_loop(0, nstage, stage, 0)

            plsc.subcore_barrier()
            pltpu.sync_copy(
                acc.at[pl.ds(sid * RTILE, RTILE)],
                out.at[pl.ds(toff + b * RPT + sid * RTILE, RTILE)])
            return bcarry

        lax.fori_loop(0, NB, bloop, 0)
        return tcarry

    lax.fori_loop(0, TPS, tloop, 0)


BN_A = 1024


def _mm_body(x_ref, wc_ref, wg_ref, bc_ref, degp_ref, out_ref):
    xb = x_ref[...]
    wg_t = wg_ref[...].T
    m = [jnp.dot(wc_ref[k].T, wg_t, preferred_element_type=_f32)
         for k in range(3)]
    hb = jnp.dot(bc_ref[...], wg_t, preferred_element_type=_f32)
    pb = degp_ref[...]
    dis = lax.rsqrt(1.0 + pb[0, :, 0:1] + pb[1, :, 0:1])
    for t in range(T):
        acc = jnp.dot(xb[t], m[1], preferred_element_type=_f32) + hb
        if t > 0:
            acc = acc + jnp.dot(xb[t - 1], m[0], preferred_element_type=_f32)
        if t < T - 1:
            acc = acc + jnp.dot(xb[t + 1], m[2], preferred_element_type=_f32)
        out_ref[t] = acc * dis


_mm_call = pl.pallas_call(
    _mm_body,
    grid=(NP // BN_A,),
    in_specs=[
        pl.BlockSpec((T, BN_A, C), lambda i: (0, i, 0)),
        pl.BlockSpec((3, C, C), lambda i: (0, 0, 0)),
        pl.BlockSpec((C, C), lambda i: (0, 0)),
        pl.BlockSpec((1, C), lambda i: (0, 0)),
        pl.BlockSpec((2, BN_A, DL), lambda i: (0, i, 0)),
    ],
    out_specs=pl.BlockSpec((T, BN_A, C), lambda i: (0, i, 0)),
    out_shape=jax.ShapeDtypeStruct((T, NP, C), _f32),
)


def _bn_body(s_ref, h_ref, degp_ref, gamma_ref, beta_ref, out_ref):
    sb = s_ref[0]
    hb = h_ref[0]
    pb = degp_ref[...]
    dis = lax.rsqrt(1.0 + pb[0, :, 0:1] + pb[1, :, 0:1])
    o = dis * (sb + hb)
    mask = lax.broadcasted_iota(_i32, (NP, 1), 0) < N
    om = jnp.where(mask, o, 0.0)
    mu = jnp.sum(om, axis=0, keepdims=True) * (1.0 / N)
    d = jnp.where(mask, o - mu, 0.0)
    var = jnp.sum(d * d, axis=0, keepdims=True) * (1.0 / N)
    scale = gamma_ref[...] * lax.rsqrt(var + 1e-5)
    out_ref[0] = jnp.maximum((o - mu) * scale + beta_ref[...], 0.0)


_bn_call = pl.pallas_call(
    _bn_body,
    grid=(T,),
    in_specs=[
        pl.BlockSpec((1, NP, C), lambda t: (t, 0, 0)),
        pl.BlockSpec((1, NP, C), lambda t: (t, 0, 0)),
        pl.BlockSpec((2, NP, DL), lambda t: (0, 0, 0)),
        pl.BlockSpec((1, C), lambda t: (0, 0)),
        pl.BlockSpec((1, C), lambda t: (0, 0)),
    ],
    out_specs=pl.BlockSpec((1, NP, C), lambda t: (t, 0, 0)),
    out_shape=jax.ShapeDtypeStruct((T, NP, C), _f32),
)


BN_D = 1000


def _out_body(x_ref, w_ref, b_ref, out_ref):
    xb = x_ref[...]
    m = xb[0]
    for t in range(1, T):
        m = m + xb[t]
    m = m * (1.0 / T)
    out_ref[...] = jnp.dot(m, w_ref[...].T, preferred_element_type=_f32) \
        + b_ref[...]


_out_call = pl.pallas_call(
    _out_body,
    grid=(N // BN_D,),
    in_specs=[
        pl.BlockSpec((T, BN_D, C), lambda i: (0, i, 0)),
        pl.BlockSpec((C, C), lambda i: (0, 0)),
        pl.BlockSpec((1, C), lambda i: (0, 0)),
    ],
    out_specs=pl.BlockSpec((BN_D, C), lambda i: (i, 0)),
    out_shape=jax.ShapeDtypeStruct((N, C), _f32),
)


def kernel(x_seq, edge_index, edge_weight, l0_wc, l0_bc, l0_wg, l0_bg,
           l0_gamma, l0_beta, l1_wc, l1_bc, l1_wg, l1_bg, l1_gamma, l1_beta,
           out_w, out_b):
    row = edge_index[0]
    col = edge_index[1]
    w = edge_weight

    brow, bcl, bw, counts2 = _scan_kernel(row, col, w)
    degp = _deg_kernel(bcl, bw, counts2).reshape(2, NP, DL)

    x = jnp.zeros((T, NP, C), _f32).at[:, :N, :].set(x_seq)
    for (wc, bc, wg, gamma, beta) in (
            (l0_wc, l0_bc, l0_wg, l0_gamma, l0_beta),
            (l1_wc, l1_bc, l1_wg, l1_gamma, l1_beta)):
        wc_r = jnp.transpose(wc, (2, 0, 1))
        hpp = _mm_call(x, wc_r, wg, bc.reshape(1, C), degp)
        s = _spmm_kernel(hpp.reshape(T * NP, C), brow, bcl, bw, counts2)
        x = _bn_call(s.reshape(T, NP, C), hpp, degp,
                     gamma.reshape(1, C), beta.reshape(1, C))
    return _out_call(x[:, :N, :], out_w, out_b.reshape(1, C))


# speedup vs baseline: 3.6928x; 1.0513x over previous
import functools

import jax
import jax.numpy as jnp
from jax import lax
from jax.experimental import pallas as pl
from jax.experimental.pallas import tpu as pltpu
from jax.experimental.pallas import tpu_sc as plsc

N = 10000
E = 320000
T = 8
C = 128
NP = 10240
NC = 2
NS = 16
RPT = NP // NS
TPS = T // NC

NB = 16
NSRC = 32
EPW = E // NSRC
RCAP = 10272
SCH = 2048
K = 32
KD = 32
DL = 16

_f32 = jnp.float32
_i32 = jnp.int32

_sc_mesh = plsc.VectorSubcoreMesh(
    core_axis_name="c", subcore_axis_name="s", num_cores=NC, num_subcores=NS)


def _extract(v0, v1, j):
    return v0[j] if j < 16 else v1[j - 16]


def _dyn_lane(v0, v1, j):
    acc = v0[0]
    for k in range(1, 16):
        acc = jnp.where(j == k, v0[k], acc)
    for k in range(16):
        acc = jnp.where(j == k + 16, v1[k], acc)
    return acc


@functools.partial(
    pl.kernel,
    out_type=(
        jax.ShapeDtypeStruct((NB * NSRC * RCAP,), _i32),
        jax.ShapeDtypeStruct((NB * NSRC * RCAP,), _i32),
        jax.ShapeDtypeStruct((NB * NSRC * RCAP,), _f32),
        jax.ShapeDtypeStruct((NSRC * NB,), _i32),
    ),
    mesh=_sc_mesh,
    compiler_params=pltpu.CompilerParams(needs_layout_passes=False),
    scratch_types=[
        pltpu.VMEM((EPW,), _i32),
        pltpu.VMEM((EPW,), _i32),
        pltpu.VMEM((EPW,), _f32),
        pltpu.VMEM((RCAP,), _i32),
        pltpu.VMEM((RCAP,), _i32),
        pltpu.VMEM((RCAP,), _f32),
        pltpu.VMEM((16,), _i32),
    ],
)
def _scan_kernel(rowv, colv, wv, brow, bcl, bw, counts,
                 srowS, scolS, swS, obrow, obcl, obw, cbuf):
    cid = lax.axis_index("c")
    sid = lax.axis_index("s")
    wid = cid * NS + sid
    pltpu.sync_copy(rowv.at[pl.ds(wid * EPW, EPW)], srowS)
    pltpu.sync_copy(colv.at[pl.ds(wid * EPW, EPW)], scolS)
    pltpu.sync_copy(wv.at[pl.ds(wid * EPW, EPW)], swS)

    zvi = jnp.zeros((16,), _i32)
    zvf = jnp.zeros((16,), _f32)
    lanes = lax.broadcasted_iota(_i32, (16,), 0)

    def bloop(b, countsv):
        lo = b * RPT

        def chunk(ci, cursor):
            base = ci * 16
            c = scolS[pl.ds(base, 16)]
            r = srowS[pl.ds(base, 16)]
            wch = swS[pl.ds(base, 16)]
            m = (c >= lo) & (c < lo + RPT)
            incl = plsc.cumsum(m.astype(_i32))
            dest = jnp.where(m, cursor + incl - 1, (RCAP - 16) + lanes)
            plsc.store_scatter(obrow, [dest], r)
            plsc.store_scatter(obcl, [dest], c - lo)
            plsc.store_scatter(obw, [dest], wch)
            return cursor + incl[15]

        cursor = lax.fori_loop(0, EPW // 16, chunk, jnp.int32(0))
        obrow[pl.ds(cursor, 16)] = zvi
        obrow[pl.ds(cursor + 16, 16)] = zvi
        obcl[pl.ds(cursor, 16)] = zvi
        obcl[pl.ds(cursor + 16, 16)] = zvi
        obw[pl.ds(cursor, 16)] = zvf
        obw[pl.ds(cursor + 16, 16)] = zvf
        roff = (b * NSRC + wid) * RCAP
        pltpu.sync_copy(obrow, brow.at[pl.ds(roff, RCAP)])
        pltpu.sync_copy(obcl, bcl.at[pl.ds(roff, RCAP)])
        pltpu.sync_copy(obw, bw.at[pl.ds(roff, RCAP)])
        return jnp.where(lanes == b, cursor, countsv)

    countsv = lax.fori_loop(0, NB, bloop, jnp.zeros((16,), _i32))
    cbuf[:] = countsv
    pltpu.sync_copy(cbuf, counts.at[pl.ds(wid * 16, 16)])


@functools.partial(
    pl.kernel,
    out_type=jax.ShapeDtypeStruct((NC * NP, DL), _f32),
    mesh=_sc_mesh,
    scratch_types=[
        pltpu.VMEM((NSRC * NB,), _i32),
        pltpu.VMEM((SCH,), _i32),
        pltpu.VMEM((SCH,), _f32),
        pltpu.VMEM((RPT, DL), _f32),
    ],
)
def _deg_kernel(bcl, bw, counts2, out, cntv, scl, sw, dacc):
    cid = lax.axis_index("c")
    b = lax.axis_index("s")
    pltpu.sync_copy(counts2, cntv)

    zv = jnp.zeros((DL,), _f32)

    def zr(i, carry):
        dacc[i, :] = zv
        return carry

    lax.fori_loop(0, RPT, zr, 0)

    def sloop(sl, carry):
        s = cid * (NSRC // NC) + sl
        cvec = cntv[pl.ds(s * 16, 16)]
        cnt = _dyn_lane(cvec, cvec, b)
        roff = (b * NSRC + s) * RCAP
        nstage = (cnt + (SCH - 1)) // SCH

        def stage(si, carry1):
            off = roff + si * SCH
            pltpu.sync_copy(bcl.at[pl.ds(off, SCH)], scl)
            pltpu.sync_copy(bw.at[pl.ds(off, SCH)], sw)
            rem = jnp.minimum(cnt - si * SCH, SCH)
            nin = (rem + (KD - 1)) // KD

            def chunk(ci, carry2):
                base = ci * KD
                c0 = scl[pl.ds(base, 16)]
                c1 = scl[pl.ds(base + 16, 16)]
                w0 = sw[pl.ds(base, 16)]
                w1 = sw[pl.ds(base + 16, 16)]
                for j in range(KD):
                    cl = _extract(c0, c1, j)
                    wj = _extract(w0, w1, j)
                    dacc[cl, :] = dacc[cl, :] + jnp.full((DL,), wj, _f32)
                return carry2

            lax.fori_loop(0, nin, chunk, 0)
            return carry1

        lax.fori_loop(0, nstage, stage, 0)
        return carry

    lax.fori_loop(0, NSRC // NC, sloop, 0)

    pltpu.sync_copy(dacc, out.at[pl.ds(cid * NP + b * RPT, RPT)])


NSLOT = 6
LEAD = 4
RTILE = RPT // NS


@functools.partial(
    pl.kernel,
    out_type=jax.ShapeDtypeStruct((T * NP, C), _f32),
    mesh=_sc_mesh,
    scratch_types=[
        pltpu.VMEM((NSRC * NB,), _i32),
        pltpu.VMEM((SCH,), _i32),
        pltpu.VMEM((SCH,), _i32),
        pltpu.VMEM((SCH,), _f32),
        [pltpu.VMEM((K,), _i32) for _ in range(NSLOT)],
        [pltpu.VMEM((K,), _i32) for _ in range(NSLOT)],
        [pltpu.VMEM((K, C), _f32) for _ in range(NSLOT)],
        pltpu.VMEM((RTILE, C), _f32),
        pltpu.VMEM_SHARED((RPT, C), _f32),
        [pltpu.SemaphoreType.DMA for _ in range(NSLOT)],
        [pltpu.SemaphoreType.DMA for _ in range(NSLOT)],
    ],
)
def _spmm_kernel(hpp, brow, bcl, bw, counts2, out,
                 cntv, srow, scl, sw, idx, colb, gbuf, zbuf, acc,
                 semg, sems):
    cid = lax.axis_index("c")
    sid = lax.axis_index("s")
    pltpu.sync_copy(counts2, cntv)

    zv = jnp.zeros((16,), _f32)

    def zb(i, carry):
        for v in range(C // 16):
            zbuf[i, pl.ds(v * 16, 16)] = zv
        return carry

    lax.fori_loop(0, RTILE, zb, 0)

    def build(k, ci, toff):
        base = ci * K
        for q in range(K // 16):
            idx[k][pl.ds(q * 16, 16)] = srow[pl.ds(base + q * 16, 16)] + toff
            colb[k][pl.ds(q * 16, 16)] = scl[pl.ds(base + q * 16, 16)]
        pltpu.async_copy(hpp.at[idx[k]], gbuf[k], semg[k])

    def scale(k, ci):
        base = ci * K
        wvecs = [sw[pl.ds(base + q * 16, 16)] for q in range(K // 16)]
        for j in range(K):
            sv = jnp.full((16,), wvecs[j // 16][j % 16], _f32)
            for v in range(C // 16):
                gbuf[k][j, pl.ds(v * 16, 16)] = \
                    gbuf[k][j, pl.ds(v * 16, 16)] * sv

    def tloop(tl, tcarry):
        t = cid * TPS + tl
        toff = t * NP

        def bloop(b, bcarry):
            pltpu.sync_copy(zbuf, acc.at[pl.ds(sid * RTILE, RTILE)])
            plsc.subcore_barrier()

            for sreg in range(2):
                s = sid * 2 + sreg
                cvec = cntv[pl.ds(s * 16, 16)]
                cnt = _dyn_lane(cvec, cvec, b)
                roff = (b * NSRC + s) * RCAP
                nstage = (cnt + (SCH - 1)) // SCH

                def stage(si, carry1):
                    off = roff + si * SCH
                    pltpu.sync_copy(brow.at[pl.ds(off, SCH)], srow)
                    pltpu.sync_copy(bcl.at[pl.ds(off, SCH)], scl)
                    pltpu.sync_copy(bw.at[pl.ds(off, SCH)], sw)
                    rem = jnp.minimum(cnt - si * SCH, SCH)
                    nin = (rem + (K - 1)) // K

                    for k in range(LEAD):
                        @pl.when(k < nin)
                        def _prime(k=k):
                            build(k, k, toff)

                    def quad(qi, carry2):
                        for k in range(NSLOT):
                            ci = qi * NSLOT + k

                            @pl.when(ci < nin)
                            def _do(k=k, ci=ci):
                                pltpu.make_async_copy(
                                    hpp.at[idx[k]], gbuf[k], semg[k]).wait()
                                scale(k, ci)
                                pltpu.async_copy(
                                    gbuf[k], acc.at[colb[k]], sems[k],
                                    add=True)
                                m = (k + LEAD) % NSLOT

                                @pl.when(ci + LEAD < nin)
                                def _prep(k=k, ci=ci, m=m):
                                    @pl.when(ci + LEAD >= NSLOT)
                                    def _drain(m=m):
                                        pltpu.make_async_copy(
                                            gbuf[m], acc.at[colb[m]],
                                            sems[m]).wait()
                                    build(m, ci + LEAD, toff)
                        return carry2

                    lax.fori_loop(0, (nin + NSLOT - 1) // NSLOT, quad, 0)
                    for k in range(NSLOT):
                        @pl.when(k < nin)
                        def _fdrain(k=k):
                            pltpu.make_async_copy(
                                gbuf[k], acc.at[colb[k]], sems[k]).wait()
                    return carry1

                lax.fori_loop(0, nstage, stage, 0)

            plsc.subcore_barrier()
            pltpu.sync_copy(
                acc.at[pl.ds(sid * RTILE, RTILE)],
                out.at[pl.ds(toff + b * RPT + sid * RTILE, RTILE)])
            return bcarry

        lax.fori_loop(0, NB, bloop, 0)
        return tcarry

    lax.fori_loop(0, TPS, tloop, 0)


BN_A = 1024


def _mm_body(x_ref, wc_ref, wg_ref, bc_ref, degp_ref, out_ref):
    xb = x_ref[...]
    wg_t = wg_ref[...].T
    m = [jnp.dot(wc_ref[k].T, wg_t, preferred_element_type=_f32)
         for k in range(3)]
    hb = jnp.dot(bc_ref[...], wg_t, preferred_element_type=_f32)
    pb = degp_ref[...]
    dis = lax.rsqrt(1.0 + pb[0, :, 0:1] + pb[1, :, 0:1])
    for t in range(T):
        acc = jnp.dot(xb[t], m[1], preferred_element_type=_f32) + hb
        if t > 0:
            acc = acc + jnp.dot(xb[t - 1], m[0], preferred_element_type=_f32)
        if t < T - 1:
            acc = acc + jnp.dot(xb[t + 1], m[2], preferred_element_type=_f32)
        out_ref[t] = acc * dis


_mm_call = pl.pallas_call(
    _mm_body,
    grid=(NP // BN_A,),
    in_specs=[
        pl.BlockSpec((T, BN_A, C), lambda i: (0, i, 0)),
        pl.BlockSpec((3, C, C), lambda i: (0, 0, 0)),
        pl.BlockSpec((C, C), lambda i: (0, 0)),
        pl.BlockSpec((1, C), lambda i: (0, 0)),
        pl.BlockSpec((2, BN_A, DL), lambda i: (0, i, 0)),
    ],
    out_specs=pl.BlockSpec((T, BN_A, C), lambda i: (0, i, 0)),
    out_shape=jax.ShapeDtypeStruct((T, NP, C), _f32),
)


def _bn_body(s_ref, h_ref, degp_ref, gamma_ref, beta_ref, out_ref):
    sb = s_ref[0]
    hb = h_ref[0]
    pb = degp_ref[...]
    dis = lax.rsqrt(1.0 + pb[0, :, 0:1] + pb[1, :, 0:1])
    o = dis * (sb + hb)
    mask = lax.broadcasted_iota(_i32, (NP, 1), 0) < N
    om = jnp.where(mask, o, 0.0)
    mu = jnp.sum(om, axis=0, keepdims=True) * (1.0 / N)
    d = jnp.where(mask, o - mu, 0.0)
    var = jnp.sum(d * d, axis=0, keepdims=True) * (1.0 / N)
    scale = gamma_ref[...] * lax.rsqrt(var + 1e-5)
    out_ref[0] = jnp.maximum((o - mu) * scale + beta_ref[...], 0.0)


_bn_call = pl.pallas_call(
    _bn_body,
    grid=(T,),
    in_specs=[
        pl.BlockSpec((1, NP, C), lambda t: (t, 0, 0)),
        pl.BlockSpec((1, NP, C), lambda t: (t, 0, 0)),
        pl.BlockSpec((2, NP, DL), lambda t: (0, 0, 0)),
        pl.BlockSpec((1, C), lambda t: (0, 0)),
        pl.BlockSpec((1, C), lambda t: (0, 0)),
    ],
    out_specs=pl.BlockSpec((1, NP, C), lambda t: (t, 0, 0)),
    out_shape=jax.ShapeDtypeStruct((T, NP, C), _f32),
)


BN_D = 1000


def _out_body(x_ref, w_ref, b_ref, out_ref):
    xb = x_ref[...]
    m = xb[0]
    for t in range(1, T):
        m = m + xb[t]
    m = m * (1.0 / T)
    out_ref[...] = jnp.dot(m, w_ref[...].T, preferred_element_type=_f32) \
        + b_ref[...]


_out_call = pl.pallas_call(
    _out_body,
    grid=(N // BN_D,),
    in_specs=[
        pl.BlockSpec((T, BN_D, C), lambda i: (0, i, 0)),
        pl.BlockSpec((C, C), lambda i: (0, 0)),
        pl.BlockSpec((1, C), lambda i: (0, 0)),
    ],
    out_specs=pl.BlockSpec((BN_D, C), lambda i: (i, 0)),
    out_shape=jax.ShapeDtypeStruct((N, C), _f32),
)


def kernel(x_seq, edge_index, edge_weight, l0_wc, l0_bc, l0_wg, l0_bg,
           l0_gamma, l0_beta, l1_wc, l1_bc, l1_wg, l1_bg, l1_gamma, l1_beta,
           out_w, out_b):
    row = edge_index[0]
    col = edge_index[1]
    w = edge_weight

    brow, bcl, bw, counts2 = _scan_kernel(row, col, w)
    degp = _deg_kernel(bcl, bw, counts2).reshape(2, NP, DL)

    x = jnp.zeros((T, NP, C), _f32).at[:, :N, :].set(x_seq)
    for (wc, bc, wg, gamma, beta) in (
            (l0_wc, l0_bc, l0_wg, l0_gamma, l0_beta),
            (l1_wc, l1_bc, l1_wg, l1_gamma, l1_beta)):
        wc_r = jnp.transpose(wc, (2, 0, 1))
        hpp = _mm_call(x, wc_r, wg, bc.reshape(1, C), degp)
        s = _spmm_kernel(hpp.reshape(T * NP, C), brow, bcl, bw, counts2)
        x = _bn_call(s.reshape(T, NP, C), hpp, degp,
                     gamma.reshape(1, C), beta.reshape(1, C))
    return _out_call(x[:, :N, :], out_w, out_b.reshape(1, C))


# speedup vs baseline: 3.7708x; 1.0211x over previous
import functools

import jax
import jax.numpy as jnp
from jax import lax
from jax.experimental import pallas as pl
from jax.experimental.pallas import tpu as pltpu
from jax.experimental.pallas import tpu_sc as plsc

N = 10000
E = 320000
T = 8
C = 128
NP = 10240
NC = 2
NS = 16
RPT = NP // NS
TPS = T // NC

NB = 16
NSRC = 32
EPW = E // NSRC
RCAP = 10272
SCH = 2048
K = 32
KD = 32
DL = 16

_f32 = jnp.float32
_i32 = jnp.int32

_sc_mesh = plsc.VectorSubcoreMesh(
    core_axis_name="c", subcore_axis_name="s", num_cores=NC, num_subcores=NS)


def _extract(v0, v1, j):
    return v0[j] if j < 16 else v1[j - 16]


def _dyn_lane(v0, v1, j):
    acc = v0[0]
    for k in range(1, 16):
        acc = jnp.where(j == k, v0[k], acc)
    for k in range(16):
        acc = jnp.where(j == k + 16, v1[k], acc)
    return acc


@functools.partial(
    pl.kernel,
    out_type=(
        jax.ShapeDtypeStruct((NB * NSRC * RCAP,), _i32),
        jax.ShapeDtypeStruct((NB * NSRC * RCAP,), _i32),
        jax.ShapeDtypeStruct((NB * NSRC * RCAP,), _f32),
        jax.ShapeDtypeStruct((NSRC * NB,), _i32),
    ),
    mesh=_sc_mesh,
    compiler_params=pltpu.CompilerParams(needs_layout_passes=False),
    scratch_types=[
        pltpu.VMEM((EPW,), _i32),
        pltpu.VMEM((EPW,), _i32),
        pltpu.VMEM((EPW,), _f32),
        pltpu.VMEM((RCAP,), _i32),
        pltpu.VMEM((RCAP,), _i32),
        pltpu.VMEM((RCAP,), _f32),
        pltpu.VMEM((16,), _i32),
    ],
)
def _scan_kernel(rowv, colv, wv, brow, bcl, bw, counts,
                 srowS, scolS, swS, obrow, obcl, obw, cbuf):
    cid = lax.axis_index("c")
    sid = lax.axis_index("s")
    wid = cid * NS + sid
    pltpu.sync_copy(rowv.at[pl.ds(wid * EPW, EPW)], srowS)
    pltpu.sync_copy(colv.at[pl.ds(wid * EPW, EPW)], scolS)
    pltpu.sync_copy(wv.at[pl.ds(wid * EPW, EPW)], swS)

    zvi = jnp.zeros((16,), _i32)
    zvf = jnp.zeros((16,), _f32)
    lanes = lax.broadcasted_iota(_i32, (16,), 0)

    def bloop(b, countsv):
        lo = b * RPT

        def chunk(ci, cursor):
            base = ci * 16
            c = scolS[pl.ds(base, 16)]
            r = srowS[pl.ds(base, 16)]
            wch = swS[pl.ds(base, 16)]
            m = (c >= lo) & (c < lo + RPT)
            incl = plsc.cumsum(m.astype(_i32))
            dest = jnp.where(m, cursor + incl - 1, (RCAP - 16) + lanes)
            plsc.store_scatter(obrow, [dest], r)
            plsc.store_scatter(obcl, [dest], c - lo)
            plsc.store_scatter(obw, [dest], wch)
            return cursor + incl[15]

        cursor = lax.fori_loop(0, EPW // 16, chunk, jnp.int32(0))
        obrow[pl.ds(cursor, 16)] = zvi
        obrow[pl.ds(cursor + 16, 16)] = zvi
        obcl[pl.ds(cursor, 16)] = zvi
        obcl[pl.ds(cursor + 16, 16)] = zvi
        obw[pl.ds(cursor, 16)] = zvf
        obw[pl.ds(cursor + 16, 16)] = zvf
        roff = (b * NSRC + wid) * RCAP
        pltpu.sync_copy(obrow, brow.at[pl.ds(roff, RCAP)])
        pltpu.sync_copy(obcl, bcl.at[pl.ds(roff, RCAP)])
        pltpu.sync_copy(obw, bw.at[pl.ds(roff, RCAP)])
        return jnp.where(lanes == b, cursor, countsv)

    countsv = lax.fori_loop(0, NB, bloop, jnp.zeros((16,), _i32))
    cbuf[:] = countsv
    pltpu.sync_copy(cbuf, counts.at[pl.ds(wid * 16, 16)])


@functools.partial(
    pl.kernel,
    out_type=jax.ShapeDtypeStruct((NC * NP, DL), _f32),
    mesh=_sc_mesh,
    scratch_types=[
        pltpu.VMEM((NSRC * NB,), _i32),
        pltpu.VMEM((SCH,), _i32),
        pltpu.VMEM((SCH,), _f32),
        pltpu.VMEM((RPT, DL), _f32),
    ],
)
def _deg_kernel(bcl, bw, counts2, out, cntv, scl, sw, dacc):
    cid = lax.axis_index("c")
    b = lax.axis_index("s")
    pltpu.sync_copy(counts2, cntv)

    zv = jnp.zeros((DL,), _f32)

    def zr(i, carry):
        dacc[i, :] = zv
        return carry

    lax.fori_loop(0, RPT, zr, 0)

    def sloop(sl, carry):
        s = cid * (NSRC // NC) + sl
        cvec = cntv[pl.ds(s * 16, 16)]
        cnt = _dyn_lane(cvec, cvec, b)
        roff = (b * NSRC + s) * RCAP
        nstage = (cnt + (SCH - 1)) // SCH

        def stage(si, carry1):
            off = roff + si * SCH
            pltpu.sync_copy(bcl.at[pl.ds(off, SCH)], scl)
            pltpu.sync_copy(bw.at[pl.ds(off, SCH)], sw)
            rem = jnp.minimum(cnt - si * SCH, SCH)
            nin = (rem + (KD - 1)) // KD

            def chunk(ci, carry2):
                base = ci * KD
                c0 = scl[pl.ds(base, 16)]
                c1 = scl[pl.ds(base + 16, 16)]
                w0 = sw[pl.ds(base, 16)]
                w1 = sw[pl.ds(base + 16, 16)]
                for j in range(KD):
                    cl = _extract(c0, c1, j)
                    wj = _extract(w0, w1, j)
                    dacc[cl, :] = dacc[cl, :] + jnp.full((DL,), wj, _f32)
                return carry2

            lax.fori_loop(0, nin, chunk, 0)
            return carry1

        lax.fori_loop(0, nstage, stage, 0)
        return carry

    lax.fori_loop(0, NSRC // NC, sloop, 0)

    pltpu.sync_copy(dacc, out.at[pl.ds(cid * NP + b * RPT, RPT)])


NSLOT = 8
LEAD = 6
RTILE = RPT // NS


@functools.partial(
    pl.kernel,
    out_type=jax.ShapeDtypeStruct((T * NP, C), _f32),
    mesh=_sc_mesh,
    scratch_types=[
        pltpu.VMEM((NSRC * NB,), _i32),
        pltpu.VMEM((SCH,), _i32),
        pltpu.VMEM((SCH,), _i32),
        pltpu.VMEM((SCH,), _f32),
        [pltpu.VMEM((K,), _i32) for _ in range(NSLOT)],
        [pltpu.VMEM((K,), _i32) for _ in range(NSLOT)],
        [pltpu.VMEM((K, C), _f32) for _ in range(NSLOT)],
        pltpu.VMEM((RTILE, C), _f32),
        pltpu.VMEM_SHARED((RPT, C), _f32),
        [pltpu.SemaphoreType.DMA for _ in range(NSLOT)],
        [pltpu.SemaphoreType.DMA for _ in range(NSLOT)],
    ],
)
def _spmm_kernel(hpp, brow, bcl, bw, counts2, out,
                 cntv, srow, scl, sw, idx, colb, gbuf, zbuf, acc,
                 semg, sems):
    cid = lax.axis_index("c")
    sid = lax.axis_index("s")
    pltpu.sync_copy(counts2, cntv)

    zv = jnp.zeros((16,), _f32)

    def zb(i, carry):
        for v in range(C // 16):
            zbuf[i, pl.ds(v * 16, 16)] = zv
        return carry

    lax.fori_loop(0, RTILE, zb, 0)

    def build(k, ci, toff):
        base = ci * K
        for q in range(K // 16):
            idx[k][pl.ds(q * 16, 16)] = srow[pl.ds(base + q * 16, 16)] + toff
            colb[k][pl.ds(q * 16, 16)] = scl[pl.ds(base + q * 16, 16)]
        pltpu.async_copy(hpp.at[idx[k]], gbuf[k], semg[k])

    def scale(k, ci):
        base = ci * K
        wvecs = [sw[pl.ds(base + q * 16, 16)] for q in range(K // 16)]
        for j in range(K):
            sv = jnp.full((16,), wvecs[j // 16][j % 16], _f32)
            for v in range(C // 16):
                gbuf[k][j, pl.ds(v * 16, 16)] = \
                    gbuf[k][j, pl.ds(v * 16, 16)] * sv

    def tloop(tl, tcarry):
        t = cid * TPS + tl
        toff = t * NP

        def bloop(b, bcarry):
            pltpu.sync_copy(zbuf, acc.at[pl.ds(sid * RTILE, RTILE)])
            plsc.subcore_barrier()

            for sreg in range(2):
                s = sid * 2 + sreg
                cvec = cntv[pl.ds(s * 16, 16)]
                cnt = _dyn_lane(cvec, cvec, b)
                roff = (b * NSRC + s) * RCAP
                nstage = (cnt + (SCH - 1)) // SCH

                def stage(si, carry1):
                    off = roff + si * SCH
                    pltpu.sync_copy(brow.at[pl.ds(off, SCH)], srow)
                    pltpu.sync_copy(bcl.at[pl.ds(off, SCH)], scl)
                    pltpu.sync_copy(bw.at[pl.ds(off, SCH)], sw)
                    rem = jnp.minimum(cnt - si * SCH, SCH)
                    nin = (rem + (K - 1)) // K

                    for k in range(LEAD):
                        @pl.when(k < nin)
                        def _prime(k=k):
                            build(k, k, toff)

                    def quad(qi, carry2):
                        for k in range(NSLOT):
                            ci = qi * NSLOT + k

                            @pl.when(ci < nin)
                            def _do(k=k, ci=ci):
                                pltpu.make_async_copy(
                                    hpp.at[idx[k]], gbuf[k], semg[k]).wait()
                                scale(k, ci)
                                pltpu.async_copy(
                                    gbuf[k], acc.at[colb[k]], sems[k],
                                    add=True)
                                m = (k + LEAD) % NSLOT

                                @pl.when(ci + LEAD < nin)
                                def _prep(k=k, ci=ci, m=m):
                                    @pl.when(ci + LEAD >= NSLOT)
                                    def _drain(m=m):
                                        pltpu.make_async_copy(
                                            gbuf[m], acc.at[colb[m]],
                                            sems[m]).wait()
                                    build(m, ci + LEAD, toff)
                        return carry2

                    lax.fori_loop(0, (nin + NSLOT - 1) // NSLOT, quad, 0)
                    for k in range(NSLOT):
                        @pl.when(k < nin)
                        def _fdrain(k=k):
                            pltpu.make_async_copy(
                                gbuf[k], acc.at[colb[k]], sems[k]).wait()
                    return carry1

                lax.fori_loop(0, nstage, stage, 0)

            plsc.subcore_barrier()
            pltpu.sync_copy(
                acc.at[pl.ds(sid * RTILE, RTILE)],
                out.at[pl.ds(toff + b * RPT + sid * RTILE, RTILE)])
            return bcarry

        lax.fori_loop(0, NB, bloop, 0)
        return tcarry

    lax.fori_loop(0, TPS, tloop, 0)


BN_A = 1024


def _mm_body(x_ref, wc_ref, wg_ref, bc_ref, degp_ref, out_ref):
    xb = x_ref[...]
    wg_t = wg_ref[...].T
    m = [jnp.dot(wc_ref[k].T, wg_t, preferred_element_type=_f32)
         for k in range(3)]
    hb = jnp.dot(bc_ref[...], wg_t, preferred_element_type=_f32)
    pb = degp_ref[...]
    dis = lax.rsqrt(1.0 + pb[0, :, 0:1] + pb[1, :, 0:1])
    for t in range(T):
        acc = jnp.dot(xb[t], m[1], preferred_element_type=_f32) + hb
        if t > 0:
            acc = acc + jnp.dot(xb[t - 1], m[0], preferred_element_type=_f32)
        if t < T - 1:
            acc = acc + jnp.dot(xb[t + 1], m[2], preferred_element_type=_f32)
        out_ref[t] = acc * dis


_mm_call = pl.pallas_call(
    _mm_body,
    grid=(NP // BN_A,),
    in_specs=[
        pl.BlockSpec((T, BN_A, C), lambda i: (0, i, 0)),
        pl.BlockSpec((3, C, C), lambda i: (0, 0, 0)),
        pl.BlockSpec((C, C), lambda i: (0, 0)),
        pl.BlockSpec((1, C), lambda i: (0, 0)),
        pl.BlockSpec((2, BN_A, DL), lambda i: (0, i, 0)),
    ],
    out_specs=pl.BlockSpec((T, BN_A, C), lambda i: (0, i, 0)),
    out_shape=jax.ShapeDtypeStruct((T, NP, C), _f32),
)


def _bn_body(s_ref, h_ref, degp_ref, gamma_ref, beta_ref, out_ref):
    sb = s_ref[0]
    hb = h_ref[0]
    pb = degp_ref[...]
    dis = lax.rsqrt(1.0 + pb[0, :, 0:1] + pb[1, :, 0:1])
    o = dis * (sb + hb)
    mask = lax.broadcasted_iota(_i32, (NP, 1), 0) < N
    om = jnp.where(mask, o, 0.0)
    mu = jnp.sum(om, axis=0, keepdims=True) * (1.0 / N)
    d = jnp.where(mask, o - mu, 0.0)
    var = jnp.sum(d * d, axis=0, keepdims=True) * (1.0 / N)
    scale = gamma_ref[...] * lax.rsqrt(var + 1e-5)
    out_ref[0] = jnp.maximum((o - mu) * scale + beta_ref[...], 0.0)


_bn_call = pl.pallas_call(
    _bn_body,
    grid=(T,),
    in_specs=[
        pl.BlockSpec((1, NP, C), lambda t: (t, 0, 0)),
        pl.BlockSpec((1, NP, C), lambda t: (t, 0, 0)),
        pl.BlockSpec((2, NP, DL), lambda t: (0, 0, 0)),
        pl.BlockSpec((1, C), lambda t: (0, 0)),
        pl.BlockSpec((1, C), lambda t: (0, 0)),
    ],
    out_specs=pl.BlockSpec((1, NP, C), lambda t: (t, 0, 0)),
    out_shape=jax.ShapeDtypeStruct((T, NP, C), _f32),
)


BN_D = 1000


def _out_body(x_ref, w_ref, b_ref, out_ref):
    xb = x_ref[...]
    m = xb[0]
    for t in range(1, T):
        m = m + xb[t]
    m = m * (1.0 / T)
    out_ref[...] = jnp.dot(m, w_ref[...].T, preferred_element_type=_f32) \
        + b_ref[...]


_out_call = pl.pallas_call(
    _out_body,
    grid=(N // BN_D,),
    in_specs=[
        pl.BlockSpec((T, BN_D, C), lambda i: (0, i, 0)),
        pl.BlockSpec((C, C), lambda i: (0, 0)),
        pl.BlockSpec((1, C), lambda i: (0, 0)),
    ],
    out_specs=pl.BlockSpec((BN_D, C), lambda i: (i, 0)),
    out_shape=jax.ShapeDtypeStruct((N, C), _f32),
)


def kernel(x_seq, edge_index, edge_weight, l0_wc, l0_bc, l0_wg, l0_bg,
           l0_gamma, l0_beta, l1_wc, l1_bc, l1_wg, l1_bg, l1_gamma, l1_beta,
           out_w, out_b):
    row = edge_index[0]
    col = edge_index[1]
    w = edge_weight

    brow, bcl, bw, counts2 = _scan_kernel(row, col, w)
    degp = _deg_kernel(bcl, bw, counts2).reshape(2, NP, DL)

    x = jnp.zeros((T, NP, C), _f32).at[:, :N, :].set(x_seq)
    for (wc, bc, wg, gamma, beta) in (
            (l0_wc, l0_bc, l0_wg, l0_gamma, l0_beta),
            (l1_wc, l1_bc, l1_wg, l1_gamma, l1_beta)):
        wc_r = jnp.transpose(wc, (2, 0, 1))
        hpp = _mm_call(x, wc_r, wg, bc.reshape(1, C), degp)
        s = _spmm_kernel(hpp.reshape(T * NP, C), brow, bcl, bw, counts2)
        x = _bn_call(s.reshape(T, NP, C), hpp, degp,
                     gamma.reshape(1, C), beta.reshape(1, C))
    return _out_call(x[:, :N, :], out_w, out_b.reshape(1, C))
